# baseline (speedup 1.0000x reference)
import functools

import jax
import jax.numpy as jnp
from jax import lax
from jax.experimental import pallas as pl
from jax.experimental.pallas import tpu as pltpu
from jax.experimental.pallas import tpu_sc as plsc

B, N, D, K = 16, 576, 256, 1024
R = B * N
TILE = 2304
NB = R // TILE


def _dist_argmin_body(x_ref, cb_ref, idx_ref, loss_ref):
    i = pl.program_id(0)
    x = x_ref[...]
    cb = cb_ref[...]
    cross = lax.dot_general(
        x, cb, (((1,), (1,)), ((), ())),
        preferred_element_type=jnp.float32)
    c2 = jnp.sum(cb * cb, axis=1)
    x2 = jnp.sum(x * x, axis=1, keepdims=True)
    d2 = jnp.maximum(x2 + c2[None, :] - 2.0 * cross, 0.0)
    idx_ref[...] = jnp.argmin(d2, axis=1).astype(jnp.int32).reshape(1, 1, TILE)
    mind2 = jnp.min(d2, axis=1)

    @pl.when(i == 0)
    def _():
        loss_ref[...] = jnp.zeros_like(loss_ref)

    loss_ref[...] = loss_ref[...] + jnp.sum(mind2)[None, None]


def _dist_argmin(x2d, codebook):
    nb = x2d.shape[0] // TILE
    return pl.pallas_call(
        _dist_argmin_body,
        grid=(nb,),
        in_specs=[
            pl.BlockSpec((TILE, D), lambda i: (i, 0)),
            pl.BlockSpec((K, D), lambda i: (0, 0)),
        ],
        out_specs=[
            pl.BlockSpec((1, 1, TILE), lambda i: (i, 0, 0)),
            pl.BlockSpec((1, 1), lambda i: (0, 0)),
        ],
        out_shape=[
            jax.ShapeDtypeStruct((nb, 1, TILE), jnp.int32),
            jax.ShapeDtypeStruct((1, 1), jnp.float32),
        ],
    )(x2d, codebook)


_NSUB = 4


def _gather_codes(codebook, idx):
    rows = idx.shape[0]
    info = plsc.get_sparse_core_info()
    nc = info.num_cores
    nw = nc * info.num_subcores
    bpw = rows // nw
    sub = bpw // _NSUB
    mesh = plsc.VectorSubcoreMesh(core_axis_name="c", subcore_axis_name="s")

    @functools.partial(
        pl.kernel, mesh=mesh,
        out_type=jax.ShapeDtypeStruct((rows, D), jnp.float32),
        scratch_types=[
            pltpu.VMEM((bpw,), jnp.int32),
            pltpu.VMEM((bpw, D), jnp.float32),
        ] + [pltpu.SemaphoreType.DMA] * (2 * _NSUB),
    )
    def k(table_hbm, idx_hbm, out_hbm, idx_v, rows_v, *sems):
        wid = lax.axis_index("s") * nc + lax.axis_index("c")
        base = wid * bpw
        pltpu.sync_copy(idx_hbm.at[pl.ds(base, bpw)], idx_v)
        gathers = [
            pltpu.async_copy(
                table_hbm.at[idx_v.at[pl.ds(j * sub, sub)]],
                rows_v.at[pl.ds(j * sub, sub)], sems[j])
            for j in range(_NSUB)
        ]
        scatters = []
        for j in range(_NSUB):
            gathers[j].wait()
            scatters.append(pltpu.async_copy(
                rows_v.at[pl.ds(j * sub, sub)],
                out_hbm.at[pl.ds(base + j * sub, sub)], sems[_NSUB + j]))
        for s in scatters:
            s.wait()

    return k(codebook, idx)


def kernel(x, codebook):
    x2d = x.reshape(R, D)
    idx3, loss_sum = _dist_argmin(x2d, codebook)
    idx = idx3.reshape(R)
    codes = _gather_codes(codebook, idx)
    quantized = codes.reshape(B, N, D)
    indices = idx.reshape(B, N)
    loss = 2.0 * loss_sum[0, 0] / jnp.float32(R * D)
    return (quantized, indices, loss)


# speedup vs baseline: 1.0072x; 1.0072x over previous
import functools

import jax
import jax.numpy as jnp
from jax import lax
from jax.experimental import pallas as pl
from jax.experimental.pallas import tpu as pltpu
from jax.experimental.pallas import tpu_sc as plsc

B, N, D, K = 16, 576, 256, 1024
R = B * N
TILE = 1152
NB = R // TILE


def _dist_argmin_body(x_ref, cb_ref, idx_ref, loss_ref):
    i = pl.program_id(0)
    x = x_ref[...]
    cb = cb_ref[...]
    cross = lax.dot_general(
        x, cb, (((1,), (1,)), ((), ())),
        preferred_element_type=jnp.float32)
    c2 = jnp.sum(cb * cb, axis=1)
    x2 = jnp.sum(x * x, axis=1, keepdims=True)
    d2 = jnp.maximum(x2 + c2[None, :] - 2.0 * cross, 0.0)
    idx_ref[...] = jnp.argmin(d2, axis=1).astype(jnp.int32).reshape(1, 1, TILE)
    mind2 = jnp.min(d2, axis=1)

    @pl.when(i == 0)
    def _():
        loss_ref[...] = jnp.zeros_like(loss_ref)

    loss_ref[...] = loss_ref[...] + jnp.sum(mind2)[None, None]


def _dist_argmin(x2d, codebook):
    nb = x2d.shape[0] // TILE
    return pl.pallas_call(
        _dist_argmin_body,
        grid=(nb,),
        in_specs=[
            pl.BlockSpec((TILE, D), lambda i: (i, 0)),
            pl.BlockSpec((K, D), lambda i: (0, 0)),
        ],
        out_specs=[
            pl.BlockSpec((1, 1, TILE), lambda i: (i, 0, 0)),
            pl.BlockSpec((1, 1), lambda i: (0, 0)),
        ],
        out_shape=[
            jax.ShapeDtypeStruct((nb, 1, TILE), jnp.int32),
            jax.ShapeDtypeStruct((1, 1), jnp.float32),
        ],
    )(x2d, codebook)


_NSUB = 4


def _gather_codes(codebook, idx):
    rows = idx.shape[0]
    info = plsc.get_sparse_core_info()
    nc = info.num_cores
    nw = nc * info.num_subcores
    bpw = rows // nw
    sub = bpw // _NSUB
    mesh = plsc.VectorSubcoreMesh(core_axis_name="c", subcore_axis_name="s")

    @functools.partial(
        pl.kernel, mesh=mesh,
        out_type=jax.ShapeDtypeStruct((rows, D), jnp.float32),
        scratch_types=[
            pltpu.VMEM((bpw,), jnp.int32),
            pltpu.VMEM((bpw, D), jnp.float32),
        ] + [pltpu.SemaphoreType.DMA] * (2 * _NSUB),
    )
    def k(table_hbm, idx_hbm, out_hbm, idx_v, rows_v, *sems):
        wid = lax.axis_index("s") * nc + lax.axis_index("c")
        base = wid * bpw
        pltpu.sync_copy(idx_hbm.at[pl.ds(base, bpw)], idx_v)
        gathers = [
            pltpu.async_copy(
                table_hbm.at[idx_v.at[pl.ds(j * sub, sub)]],
                rows_v.at[pl.ds(j * sub, sub)], sems[j])
            for j in range(_NSUB)
        ]
        scatters = []
        for j in range(_NSUB):
            gathers[j].wait()
            scatters.append(pltpu.async_copy(
                rows_v.at[pl.ds(j * sub, sub)],
                out_hbm.at[pl.ds(base + j * sub, sub)], sems[_NSUB + j]))
        for s in scatters:
            s.wait()

    return k(codebook, idx)


def kernel(x, codebook):
    x2d = x.reshape(R, D)
    idx3, loss_sum = _dist_argmin(x2d, codebook)
    idx = idx3.reshape(R)
    codes = _gather_codes(codebook, idx)
    quantized = codes.reshape(B, N, D)
    indices = idx.reshape(B, N)
    loss = 2.0 * loss_sum[0, 0] / jnp.float32(R * D)
    return (quantized, indices, loss)


# speedup vs baseline: 1.0802x; 1.0725x over previous
import functools

import jax
import jax.numpy as jnp
from jax import lax
from jax.experimental import pallas as pl
from jax.experimental.pallas import tpu as pltpu
from jax.experimental.pallas import tpu_sc as plsc

B, N, D, K = 16, 576, 256, 1024
R = B * N
TILE = 1536
NB = R // TILE
HALF = NB // 2
R_SC = HALF * TILE


def _dist_argmin_body(x_ref, cb_ref, idx_ref, loss_ref, codes_ref):
    i = pl.program_id(0)
    x = x_ref[...]
    cb = cb_ref[...]
    cross = lax.dot_general(
        x, cb, (((1,), (1,)), ((), ())),
        preferred_element_type=jnp.float32)
    c2 = jnp.sum(cb * cb, axis=1)
    x2 = jnp.sum(x * x, axis=1, keepdims=True)
    d2 = jnp.maximum(x2 + c2[None, :] - 2.0 * cross, 0.0)
    idxv = jnp.argmin(d2, axis=1).astype(jnp.int32)
    idx_ref[...] = idxv.reshape(1, 1, TILE)
    mind2 = jnp.min(d2, axis=1)

    @pl.when(i == 0)
    def _():
        loss_ref[...] = jnp.zeros_like(loss_ref)

    loss_ref[...] = loss_ref[...] + jnp.sum(mind2)[None, None]

    @pl.when(i >= HALF)
    def _():
        kk = lax.broadcasted_iota(jnp.int32, (TILE, K), 1)
        onehot = (kk == idxv[:, None]).astype(jnp.float32)
        codes_ref[...] = lax.dot_general(
            onehot, cb, (((1,), (0,)), ((), ())),
            preferred_element_type=jnp.float32)


def _dist_argmin(x2d, codebook):
    nb = x2d.shape[0] // TILE
    return pl.pallas_call(
        _dist_argmin_body,
        grid=(nb,),
        in_specs=[
            pl.BlockSpec((TILE, D), lambda i: (i, 0)),
            pl.BlockSpec((K, D), lambda i: (0, 0)),
        ],
        out_specs=[
            pl.BlockSpec((1, 1, TILE), lambda i: (i, 0, 0)),
            pl.BlockSpec((1, 1), lambda i: (0, 0)),
            pl.BlockSpec((TILE, D), lambda i: (i, 0)),
        ],
        out_shape=[
            jax.ShapeDtypeStruct((nb, 1, TILE), jnp.int32),
            jax.ShapeDtypeStruct((1, 1), jnp.float32),
            jax.ShapeDtypeStruct((nb * TILE, D), jnp.float32),
        ],
    )(x2d, codebook)


def _gather_codes(codebook, idx):
    rows = idx.shape[0]
    info = plsc.get_sparse_core_info()
    nc = info.num_cores
    nw = nc * info.num_subcores
    bpw = rows // nw
    nsub = 4 if bpw % 32 == 0 else 2
    sub = bpw // nsub
    mesh = plsc.VectorSubcoreMesh(core_axis_name="c", subcore_axis_name="s")

    @functools.partial(
        pl.kernel, mesh=mesh,
        out_type=jax.ShapeDtypeStruct((rows, D), jnp.float32),
        scratch_types=[
            pltpu.VMEM((bpw,), jnp.int32),
            pltpu.VMEM((bpw, D), jnp.float32),
        ] + [pltpu.SemaphoreType.DMA] * (2 * nsub),
    )
    def k(table_hbm, idx_hbm, out_hbm, idx_v, rows_v, *sems):
        wid = lax.axis_index("s") * nc + lax.axis_index("c")
        base = wid * bpw
        pltpu.sync_copy(idx_hbm.at[pl.ds(base, bpw)], idx_v)
        gathers = [
            pltpu.async_copy(
                table_hbm.at[idx_v.at[pl.ds(j * sub, sub)]],
                rows_v.at[pl.ds(j * sub, sub)], sems[j])
            for j in range(nsub)
        ]
        scatters = []
        for j in range(nsub):
            gathers[j].wait()
            scatters.append(pltpu.async_copy(
                rows_v.at[pl.ds(j * sub, sub)],
                out_hbm.at[pl.ds(base + j * sub, sub)], sems[nsub + j]))
        for s in scatters:
            s.wait()

    return k(codebook, idx)


def kernel(x, codebook):
    x2d = x.reshape(R, D)
    idx3, loss_sum, codes_tc = _dist_argmin(x2d, codebook)
    idx = idx3.reshape(R)
    codes_sc = _gather_codes(codebook, lax.slice(idx, (0,), (R_SC,)))
    codes = lax.dynamic_update_slice(codes_tc, codes_sc, (0, 0))
    quantized = codes.reshape(B, N, D)
    indices = idx.reshape(B, N)
    loss = 2.0 * loss_sum[0, 0] / jnp.float32(R * D)
    return (quantized, indices, loss)
